# baseline (speedup 1.0000x reference)
import jax
import jax.numpy as jnp
from jax import lax
from jax.experimental import pallas as pl
from jax.experimental.pallas import tpu as pltpu
from jax.experimental.pallas import tpu_sc as plsc

N = 320000
D_FEAT = 128
BATCH = 32

NUM_CORES = 2
NUM_SUBCORES = 16
ROWS_PER_CORE = N // NUM_CORES

CHUNK = 128
SUB = 128
SUBS_PER_CHUNK = CHUNK // SUB
CHUNKS_PER_CORE = ROWS_PER_CORE // CHUNK
BASE_CHUNKS = CHUNKS_PER_CORE // NUM_SUBCORES
EXTRA = CHUNKS_PER_CORE - BASE_CHUNKS * NUM_SUBCORES
NBUF = 7


def _body(feat_hbm, idx_hbm, sums_hbm, cnts_hbm,
          rows_v, idx_v, zeros_v, cnt_local, iota32, acc_sh, cnt_sh,
          ldsem, scsem):
    c = lax.axis_index("c")
    s = lax.axis_index("s")

    zero = jnp.zeros((16,), jnp.float32)
    ones16 = jnp.full((16,), 1.0, jnp.float32)
    lane = lax.iota(jnp.int32, 16)

    for i in range(BATCH):
        cnt_local[i, :] = zero
    iota32[pl.ds(0, 16)] = lane
    iota32[pl.ds(16, 16)] = lane + 16

    @pl.when(s == 0)
    def _init():
        for i in range(BATCH):
            for j in range(D_FEAT // 16):
                zeros_v[i, pl.ds(16 * j, 16)] = zero
        pltpu.sync_copy(zeros_v, acc_sh)
        pltpu.sync_copy(zeros_v.at[:, pl.ds(0, 16)], cnt_sh)

    plsc.subcore_barrier()

    base = s * BASE_CHUNKS + jnp.maximum(s - (NUM_SUBCORES - EXTRA), 0)
    nch = BASE_CHUNKS + jnp.where(s >= NUM_SUBCORES - EXTRA, 1, 0)

    def _load_slices(ch, b):
        r0 = c * ROWS_PER_CORE + ch * CHUNK
        return (
            (feat_hbm.at[pl.ds(r0, CHUNK)], rows_v.at[b]),
            (idx_hbm.at[pl.ds(r0 // SUB, SUBS_PER_CHUNK)], idx_v.at[b]),
        )

    def _issue_loads(ch, b):
        for src, dst in _load_slices(ch, b):
            pltpu.async_copy(src, dst, ldsem)

    def _wait_loads(ch, b):
        for src, dst in _load_slices(ch, b):
            pltpu.make_async_copy(src, dst, ldsem).wait()

    def _scatter_copies(b):
        for j in range(SUBS_PER_CHUNK):
            yield (rows_v.at[b, pl.ds(j * SUB, SUB)], acc_sh.at[idx_v.at[b, j]])

    def _issue_scatters(b):
        for src, dst in _scatter_copies(b):
            pltpu.async_copy(src, dst, scsem, add=True)

    def _drain_scatters(b):
        for src, dst in _scatter_copies(b):
            pltpu.make_async_copy(src, dst, scsem).wait()

    for i in range(NBUF - 1):
        _issue_loads(base + i, i)

    def chunk_body(k, carry):
        b = lax.rem(k, NBUF)
        bn = lax.rem(k + NBUF - 1, NBUF)
        ch = base + k
        _wait_loads(ch, b)

        for j in range(SUBS_PER_CHUNK):
            for g in range(SUB // 16):
                v = idx_v[b, j, pl.ds(16 * g, 16)]
                plsc.addupdate_scatter(cnt_local, [v, lane], ones16)

        @pl.when(k > 0)
        def _drain_prev():
            _drain_scatters(bn)

        @pl.when(k + NBUF - 1 < nch)
        def _prefetch():
            _issue_loads(ch + NBUF - 1, bn)

        _issue_scatters(b)
        return carry

    lax.fori_loop(0, nch, chunk_body, 0)
    _drain_scatters(lax.rem(nch - 1, NBUF))

    pltpu.sync_copy(cnt_local, cnt_sh.at[iota32], add=True)

    plsc.subcore_barrier()

    @pl.when(s == 0)
    def _publish():
        pltpu.sync_copy(acc_sh, sums_hbm.at[c])
        pltpu.sync_copy(cnt_sh, cnts_hbm.at[c])


def _combine_body(sums_ref, cnts_ref, out_ref):
    sums = sums_ref[0] + sums_ref[1]
    cnts = cnts_ref[0] + cnts_ref[1]
    denom = jnp.maximum(jnp.sum(cnts, axis=1, keepdims=True), 1.0)
    out_ref[...] = sums / denom


def kernel(features, batch_idx):
    idx2d = batch_idx.astype(jnp.int32).reshape(N // SUB, SUB)
    mesh = plsc.VectorSubcoreMesh(core_axis_name="c", subcore_axis_name="s")
    run = pl.kernel(
        _body,
        out_type=(
            jax.ShapeDtypeStruct((NUM_CORES, BATCH, D_FEAT), jnp.float32),
            jax.ShapeDtypeStruct((NUM_CORES, BATCH, 16), jnp.float32),
        ),
        mesh=mesh,
        compiler_params=pltpu.CompilerParams(use_tc_tiling_on_sc=False, needs_layout_passes=False),
        scratch_types=[
            pltpu.VMEM((NBUF, CHUNK, D_FEAT), jnp.float32),
            pltpu.VMEM((NBUF, SUBS_PER_CHUNK, SUB), jnp.int32),
            pltpu.VMEM((BATCH, D_FEAT), jnp.float32),
            pltpu.VMEM((BATCH, 16), jnp.float32),
            pltpu.VMEM((BATCH,), jnp.int32),
            pltpu.VMEM_SHARED((BATCH, D_FEAT), jnp.float32),
            pltpu.VMEM_SHARED((BATCH, 16), jnp.float32),
            pltpu.SemaphoreType.DMA,
            pltpu.SemaphoreType.DMA,
        ],
    )
    sums, cnts = run(features, idx2d)
    return pl.pallas_call(
        _combine_body,
        out_shape=jax.ShapeDtypeStruct((BATCH, D_FEAT), jnp.float32),
    )(sums, cnts)


# speedup vs baseline: 1.0018x; 1.0018x over previous
import jax
import jax.numpy as jnp
from jax import lax
from jax.experimental import pallas as pl
from jax.experimental.pallas import tpu as pltpu
from jax.experimental.pallas import tpu_sc as plsc

N = 320000
D_FEAT = 128
BATCH = 32

NUM_CORES = 2
NUM_SUBCORES = 16
ROWS_PER_CORE = N // NUM_CORES

CHUNK = 128
SUB = 128
SUBS_PER_CHUNK = CHUNK // SUB
CHUNKS_PER_CORE = ROWS_PER_CORE // CHUNK
BASE_CHUNKS = CHUNKS_PER_CORE // NUM_SUBCORES
EXTRA = CHUNKS_PER_CORE - BASE_CHUNKS * NUM_SUBCORES
NBUF = 6


def _body(feat_hbm, idx_hbm, sums_hbm, cnts_hbm,
          rows_v, idx_v, zeros_v, cnt_local, iota32, acc_sh, cnt_sh,
          ldsem, scsem):
    c = lax.axis_index("c")
    s = lax.axis_index("s")

    zero = jnp.zeros((16,), jnp.float32)
    ones16 = jnp.full((16,), 1.0, jnp.float32)
    lane = lax.iota(jnp.int32, 16)

    for i in range(BATCH):
        cnt_local[i, :] = zero
    iota32[pl.ds(0, 16)] = lane
    iota32[pl.ds(16, 16)] = lane + 16

    @pl.when(s == 0)
    def _init():
        for i in range(BATCH):
            for j in range(D_FEAT // 16):
                zeros_v[i, pl.ds(16 * j, 16)] = zero
        pltpu.sync_copy(zeros_v, acc_sh)
        pltpu.sync_copy(zeros_v.at[:, pl.ds(0, 16)], cnt_sh)

    plsc.subcore_barrier()

    base = s * BASE_CHUNKS + jnp.maximum(s - (NUM_SUBCORES - EXTRA), 0)
    nch = BASE_CHUNKS + jnp.where(s >= NUM_SUBCORES - EXTRA, 1, 0)

    def _load_slices(ch, b):
        r0 = c * ROWS_PER_CORE + ch * CHUNK
        return (
            (feat_hbm.at[pl.ds(r0, CHUNK)], rows_v.at[b]),
            (idx_hbm.at[pl.ds(r0 // SUB, SUBS_PER_CHUNK)], idx_v.at[b]),
        )

    def _issue_loads(ch, b):
        for src, dst in _load_slices(ch, b):
            pltpu.async_copy(src, dst, ldsem)

    def _wait_loads(ch, b):
        for src, dst in _load_slices(ch, b):
            pltpu.make_async_copy(src, dst, ldsem).wait()

    def _scatter_copies(b):
        for j in range(SUBS_PER_CHUNK):
            yield (rows_v.at[b, pl.ds(j * SUB, SUB)], acc_sh.at[idx_v.at[b, j]])

    def _issue_scatters(b):
        for src, dst in _scatter_copies(b):
            pltpu.async_copy(src, dst, scsem, add=True)

    def _drain_scatters(b):
        for src, dst in _scatter_copies(b):
            pltpu.make_async_copy(src, dst, scsem).wait()

    for i in range(NBUF - 1):
        _issue_loads(base + i, i)

    def chunk_body(k, carry):
        b = lax.rem(k, NBUF)
        bn = lax.rem(k + NBUF - 1, NBUF)
        ch = base + k
        _wait_loads(ch, b)

        for j in range(SUBS_PER_CHUNK):
            for g in range(SUB // 16):
                v = idx_v[b, j, pl.ds(16 * g, 16)]
                plsc.addupdate_scatter(cnt_local, [v, lane], ones16)

        @pl.when(k > 0)
        def _drain_prev():
            _drain_scatters(bn)

        @pl.when(k + NBUF - 1 < nch)
        def _prefetch():
            _issue_loads(ch + NBUF - 1, bn)

        _issue_scatters(b)
        return carry

    lax.fori_loop(0, nch, chunk_body, 0)
    _drain_scatters(lax.rem(nch - 1, NBUF))

    pltpu.sync_copy(cnt_local, cnt_sh.at[iota32], add=True)

    plsc.subcore_barrier()

    @pl.when(s == 0)
    def _publish():
        pltpu.sync_copy(acc_sh, sums_hbm.at[c])
        pltpu.sync_copy(cnt_sh, cnts_hbm.at[c])


def _combine_body(sums_ref, cnts_ref, out_ref):
    sums = sums_ref[0] + sums_ref[1]
    cnts = cnts_ref[0] + cnts_ref[1]
    denom = jnp.maximum(jnp.sum(cnts, axis=1, keepdims=True), 1.0)
    out_ref[...] = sums / denom


def kernel(features, batch_idx):
    idx2d = batch_idx.astype(jnp.int32).reshape(N // SUB, SUB)
    mesh = plsc.VectorSubcoreMesh(core_axis_name="c", subcore_axis_name="s")
    run = pl.kernel(
        _body,
        out_type=(
            jax.ShapeDtypeStruct((NUM_CORES, BATCH, D_FEAT), jnp.float32),
            jax.ShapeDtypeStruct((NUM_CORES, BATCH, 16), jnp.float32),
        ),
        mesh=mesh,
        compiler_params=pltpu.CompilerParams(use_tc_tiling_on_sc=False, needs_layout_passes=False),
        scratch_types=[
            pltpu.VMEM((NBUF, CHUNK, D_FEAT), jnp.float32),
            pltpu.VMEM((NBUF, SUBS_PER_CHUNK, SUB), jnp.int32),
            pltpu.VMEM((BATCH, D_FEAT), jnp.float32),
            pltpu.VMEM((BATCH, 16), jnp.float32),
            pltpu.VMEM((BATCH,), jnp.int32),
            pltpu.VMEM_SHARED((BATCH, D_FEAT), jnp.float32),
            pltpu.VMEM_SHARED((BATCH, 16), jnp.float32),
            pltpu.SemaphoreType.DMA,
            pltpu.SemaphoreType.DMA,
        ],
    )
    sums, cnts = run(features, idx2d)
    return pl.pallas_call(
        _combine_body,
        out_shape=jax.ShapeDtypeStruct((BATCH, D_FEAT), jnp.float32),
    )(sums, cnts)


# speedup vs baseline: 1.0031x; 1.0013x over previous
import jax
import jax.numpy as jnp
from jax import lax
from jax.experimental import pallas as pl
from jax.experimental.pallas import tpu as pltpu
from jax.experimental.pallas import tpu_sc as plsc

N = 320000
D_FEAT = 128
BATCH = 32

NUM_CORES = 2
NUM_SUBCORES = 16
ROWS_PER_CORE = N // NUM_CORES

CHUNK = 64
SUB = 64
SUBS_PER_CHUNK = CHUNK // SUB
CHUNKS_PER_CORE = ROWS_PER_CORE // CHUNK
BASE_CHUNKS = CHUNKS_PER_CORE // NUM_SUBCORES
EXTRA = CHUNKS_PER_CORE - BASE_CHUNKS * NUM_SUBCORES
NBUF = 12


def _body(feat_hbm, idx_hbm, sums_hbm, cnts_hbm,
          rows_v, idx_v, zeros_v, cnt_local, iota32, acc_sh, cnt_sh,
          ldsem, scsem):
    c = lax.axis_index("c")
    s = lax.axis_index("s")

    zero = jnp.zeros((16,), jnp.float32)
    ones16 = jnp.full((16,), 1.0, jnp.float32)
    lane = lax.iota(jnp.int32, 16)

    for i in range(BATCH):
        cnt_local[i, :] = zero
    iota32[pl.ds(0, 16)] = lane
    iota32[pl.ds(16, 16)] = lane + 16

    @pl.when(s == 0)
    def _init():
        for i in range(BATCH):
            for j in range(D_FEAT // 16):
                zeros_v[i, pl.ds(16 * j, 16)] = zero
        pltpu.sync_copy(zeros_v, acc_sh)
        pltpu.sync_copy(zeros_v.at[:, pl.ds(0, 16)], cnt_sh)

    plsc.subcore_barrier()

    base = s * BASE_CHUNKS + jnp.maximum(s - (NUM_SUBCORES - EXTRA), 0)
    nch = BASE_CHUNKS + jnp.where(s >= NUM_SUBCORES - EXTRA, 1, 0)

    def _load_slices(ch, b):
        r0 = c * ROWS_PER_CORE + ch * CHUNK
        return (
            (feat_hbm.at[pl.ds(r0, CHUNK)], rows_v.at[b]),
            (idx_hbm.at[pl.ds(r0 // SUB, SUBS_PER_CHUNK)], idx_v.at[b]),
        )

    def _issue_loads(ch, b):
        for src, dst in _load_slices(ch, b):
            pltpu.async_copy(src, dst, ldsem)

    def _wait_loads(ch, b):
        for src, dst in _load_slices(ch, b):
            pltpu.make_async_copy(src, dst, ldsem).wait()

    def _scatter_copies(b):
        for j in range(SUBS_PER_CHUNK):
            yield (rows_v.at[b, pl.ds(j * SUB, SUB)], acc_sh.at[idx_v.at[b, j]])

    def _issue_scatters(b):
        for src, dst in _scatter_copies(b):
            pltpu.async_copy(src, dst, scsem, add=True)

    def _drain_scatters(b):
        for src, dst in _scatter_copies(b):
            pltpu.make_async_copy(src, dst, scsem).wait()

    for i in range(NBUF - 1):
        _issue_loads(base + i, i)

    def chunk_body(k, carry):
        b = lax.rem(k, NBUF)
        bn = lax.rem(k + NBUF - 1, NBUF)
        ch = base + k
        _wait_loads(ch, b)

        for j in range(SUBS_PER_CHUNK):
            for g in range(SUB // 16):
                v = idx_v[b, j, pl.ds(16 * g, 16)]
                plsc.addupdate_scatter(cnt_local, [v, lane], ones16)

        @pl.when(k > 0)
        def _drain_prev():
            _drain_scatters(bn)

        @pl.when(k + NBUF - 1 < nch)
        def _prefetch():
            _issue_loads(ch + NBUF - 1, bn)

        _issue_scatters(b)
        return carry

    lax.fori_loop(0, nch, chunk_body, 0)
    _drain_scatters(lax.rem(nch - 1, NBUF))

    pltpu.sync_copy(cnt_local, cnt_sh.at[iota32], add=True)

    plsc.subcore_barrier()

    @pl.when(s == 0)
    def _publish():
        pltpu.sync_copy(acc_sh, sums_hbm.at[c])
        pltpu.sync_copy(cnt_sh, cnts_hbm.at[c])


def _combine_body(sums_ref, cnts_ref, out_ref):
    sums = sums_ref[0] + sums_ref[1]
    cnts = cnts_ref[0] + cnts_ref[1]
    denom = jnp.maximum(jnp.sum(cnts, axis=1, keepdims=True), 1.0)
    out_ref[...] = sums / denom


def kernel(features, batch_idx):
    idx2d = batch_idx.astype(jnp.int32).reshape(N // SUB, SUB)
    mesh = plsc.VectorSubcoreMesh(core_axis_name="c", subcore_axis_name="s")
    run = pl.kernel(
        _body,
        out_type=(
            jax.ShapeDtypeStruct((NUM_CORES, BATCH, D_FEAT), jnp.float32),
            jax.ShapeDtypeStruct((NUM_CORES, BATCH, 16), jnp.float32),
        ),
        mesh=mesh,
        compiler_params=pltpu.CompilerParams(use_tc_tiling_on_sc=False, needs_layout_passes=False),
        scratch_types=[
            pltpu.VMEM((NBUF, CHUNK, D_FEAT), jnp.float32),
            pltpu.VMEM((NBUF, SUBS_PER_CHUNK, SUB), jnp.int32),
            pltpu.VMEM((BATCH, D_FEAT), jnp.float32),
            pltpu.VMEM((BATCH, 16), jnp.float32),
            pltpu.VMEM((BATCH,), jnp.int32),
            pltpu.VMEM_SHARED((BATCH, D_FEAT), jnp.float32),
            pltpu.VMEM_SHARED((BATCH, 16), jnp.float32),
            pltpu.SemaphoreType.DMA,
            pltpu.SemaphoreType.DMA,
        ],
    )
    sums, cnts = run(features, idx2d)
    return pl.pallas_call(
        _combine_body,
        out_shape=jax.ShapeDtypeStruct((BATCH, D_FEAT), jnp.float32),
    )(sums, cnts)
